# baseline (speedup 1.0000x reference)
import functools

import jax
import jax.numpy as jnp
from jax import lax
from jax.experimental import pallas as pl
from jax.experimental.pallas import tpu as pltpu
from jax.experimental.pallas import tpu_sc as plsc

N_E = 2048
E_DIM = 32
P = 8
BETA = 0.25
GPB = 2
NC = 2
NS = 16
CHUNK = 128


def _argmin_block_kernel(z_ref, e_ref, idx_ref, e2_ref, esq_ref):
    hw = z_ref.shape[2]
    cols = GPB * hw
    zbf = z_ref[0]
    zb = jnp.concatenate(
        [zbf[g * E_DIM:(g + 1) * E_DIM, :] for g in range(GPB)], axis=1)
    step = pl.program_id(0)

    @pl.when(step == 0)
    def _init():
        e = e_ref[...]
        e2_ref[...] = e + e
        esq_ref[...] = jnp.sum(e * e, axis=1, keepdims=True)

    e_sq = esq_ref[...]
    z_sq = jnp.sum(zb * zb, axis=0, keepdims=True)
    mm2 = jax.lax.dot_general(
        e2_ref[...], zb, (((1,), (0,)), ((), ())))
    d = (z_sq + e_sq) - mm2

    dmin = jnp.min(d, axis=0, keepdims=True)
    riota = jax.lax.broadcasted_iota(jnp.int32, (N_E, cols), 0)
    cand = jnp.where(d == dmin, riota, N_E)
    idx_ref[...] = jnp.min(cand, axis=0, keepdims=True).reshape(1, 1, cols)


def _sc_gather_body(table_hbm, idx_hbm, out_hbm, idx_v, rows_v, sem):
    wid = lax.axis_index("s") * NC + lax.axis_index("c")
    rows_per_w = CHUNK * 8
    pltpu.sync_copy(idx_hbm.at[pl.ds(wid * 8, 8)], idx_v)
    copies = [
        pltpu.async_copy(
            table_hbm.at[idx_v.at[j]],
            rows_v.at[pl.ds(j * CHUNK, CHUNK)],
            sem)
        for j in range(8)
    ]
    for c in copies:
        c.wait()
    pltpu.sync_copy(rows_v, out_hbm.at[pl.ds(wid * rows_per_w, rows_per_w)])


def _ste_block_kernel(z_ref, q_ref, zq_ref, loss_ref):
    hw = z_ref.shape[2]
    zbf = z_ref[0]
    zb = jnp.concatenate(
        [zbf[g * E_DIM:(g + 1) * E_DIM, :] for g in range(GPB)], axis=1)
    step = pl.program_id(0)

    @pl.when(step == 0)
    def _init():
        loss_ref[...] = jnp.zeros((1, 1), jnp.float32)

    zq_t = q_ref[...].T
    diff = zq_t - zb
    loss_ref[...] += jnp.sum(diff * diff).reshape(1, 1)
    out = zb + diff
    zq_ref[...] = jnp.concatenate(
        [out[:, g * hw:(g + 1) * hw] for g in range(GPB)],
        axis=0).reshape(z_ref.shape)


@functools.partial(jax.jit, static_argnames=())
def kernel(z, embedding_weight):
    b, c, h, w = z.shape
    new_c = c // P
    hw = h * w
    grid = (b * P) // GPB
    cols = GPB * hw
    rows = b * P * hw
    z3 = z.reshape(b, c, hw)
    blocks_per_b = P // GPB

    idx16 = pl.pallas_call(
        _argmin_block_kernel,
        grid=(grid,),
        in_specs=[
            pl.BlockSpec((1, GPB * new_c, hw),
                         lambda i: (i // blocks_per_b, i % blocks_per_b, 0)),
            pl.BlockSpec((N_E, E_DIM), lambda i: (0, 0)),
        ],
        out_specs=pl.BlockSpec((1, 1, cols), lambda i: (i, 0, 0)),
        out_shape=jax.ShapeDtypeStruct((grid, 1, cols), jnp.int32),
        scratch_shapes=[
            pltpu.VMEM((N_E, E_DIM), jnp.float32),
            pltpu.VMEM((N_E, 1), jnp.float32),
        ],
    )(z3, embedding_weight)

    idx2d = idx16.reshape(rows // CHUNK, CHUNK)

    mesh = plsc.VectorSubcoreMesh(core_axis_name="c", subcore_axis_name="s")
    zq_flat = pl.kernel(
        _sc_gather_body,
        out_type=jax.ShapeDtypeStruct((rows, E_DIM), jnp.float32),
        mesh=mesh,
        compiler_params=pltpu.CompilerParams(use_tc_tiling_on_sc=False),
        scratch_types=[
            pltpu.VMEM((8, CHUNK), jnp.int32),
            pltpu.VMEM((8 * CHUNK, E_DIM), jnp.float32),
            pltpu.SemaphoreType.DMA,
        ],
    )(embedding_weight, idx2d)

    zq, loss_sum = pl.pallas_call(
        _ste_block_kernel,
        grid=(grid,),
        in_specs=[
            pl.BlockSpec((1, GPB * new_c, hw),
                         lambda i: (i // blocks_per_b, i % blocks_per_b, 0)),
            pl.BlockSpec((cols, E_DIM), lambda i: (i, 0)),
        ],
        out_specs=[
            pl.BlockSpec((1, GPB * new_c, hw),
                         lambda i: (i // blocks_per_b, i % blocks_per_b, 0)),
            pl.BlockSpec((1, 1), lambda i: (0, 0)),
        ],
        out_shape=[
            jax.ShapeDtypeStruct((b, c, hw), jnp.float32),
            jax.ShapeDtypeStruct((1, 1), jnp.float32),
        ],
    )(z3, zq_flat)

    n_el = b * c * h * w
    m = loss_sum[0, 0] / n_el
    loss = m + BETA * m
    return (zq.reshape(b, c, h, w), loss)


# speedup vs baseline: 1.3086x; 1.3086x over previous
import functools

import jax
import jax.numpy as jnp
from jax.experimental import pallas as pl
from jax.experimental.pallas import tpu as pltpu

N_E = 2048
E_DIM = 32
P = 8
BETA = 0.25
HI = 16
LO = 128
GPB = 2


def _vq_block_kernel(z_ref, e_ref, e2t_ref, zq_ref, loss_ref, e2_ref, esq_ref):
    hw = z_ref.shape[2]
    cols = GPB * hw
    zbf = z_ref[0]
    zb = jnp.concatenate(
        [zbf[g * E_DIM:(g + 1) * E_DIM, :] for g in range(GPB)], axis=1)
    step = pl.program_id(0)

    @pl.when(step == 0)
    def _init():
        e = e_ref[...]
        e2_ref[...] = e + e
        esq_ref[...] = jnp.sum(e * e, axis=1, keepdims=True)
        loss_ref[...] = jnp.zeros((1, 1), jnp.float32)

    e_sq = esq_ref[...]
    z_sq = jnp.sum(zb * zb, axis=0, keepdims=True)
    mm2 = jax.lax.dot_general(
        e2_ref[...], zb, (((1,), (0,)), ((), ())))
    d = (z_sq + e_sq) - mm2

    dmin = jnp.min(d, axis=0, keepdims=True)
    riota = jax.lax.broadcasted_iota(jnp.int32, (N_E, cols), 0)
    cand = jnp.where(d == dmin, riota, N_E)
    idx = jnp.min(cand, axis=0, keepdims=True)

    lo = idx & (LO - 1)
    hi = idx >> 7
    liota = jax.lax.broadcasted_iota(jnp.int32, (LO, cols), 0)
    ohlo = jnp.where(liota == lo, 1.0, 0.0).astype(jnp.bfloat16)
    t = jax.lax.dot_general(
        e2t_ref[...], ohlo, (((1,), (0,)), ((), ())),
        preferred_element_type=jnp.float32)
    siota = jax.lax.broadcasted_iota(jnp.int32, (HI * E_DIM, cols), 0) >> 5
    pm = jnp.where(siota == hi, t, 0.0)
    acc = pm[0:E_DIM, :]
    for h in range(1, HI):
        acc = acc + pm[h * E_DIM:(h + 1) * E_DIM, :]

    diff = acc - zb
    loss_ref[...] += jnp.sum(diff * diff).reshape(1, 1)
    out = zb + diff
    zq_ref[...] = jnp.concatenate(
        [out[:, g * hw:(g + 1) * hw] for g in range(GPB)],
        axis=0).reshape(z_ref.shape)


@functools.partial(jax.jit, static_argnames=())
def kernel(z, embedding_weight):
    b, c, h, w = z.shape
    new_c = c // P
    hw = h * w
    grid = (b * P) // GPB

    e2t = (embedding_weight.reshape(HI, LO, E_DIM)
           .transpose(0, 2, 1)
           .reshape(HI * E_DIM, LO)
           .astype(jnp.bfloat16))
    z3 = z.reshape(b, c, hw)
    blocks_per_b = P // GPB

    zq, loss_sum = pl.pallas_call(
        _vq_block_kernel,
        grid=(grid,),
        in_specs=[
            pl.BlockSpec((1, GPB * new_c, hw),
                         lambda i: (i // blocks_per_b, i % blocks_per_b, 0)),
            pl.BlockSpec((N_E, E_DIM), lambda i: (0, 0)),
            pl.BlockSpec((HI * E_DIM, LO), lambda i: (0, 0)),
        ],
        out_specs=[
            pl.BlockSpec((1, GPB * new_c, hw),
                         lambda i: (i // blocks_per_b, i % blocks_per_b, 0)),
            pl.BlockSpec((1, 1), lambda i: (0, 0)),
        ],
        out_shape=[
            jax.ShapeDtypeStruct((b, c, hw), jnp.float32),
            jax.ShapeDtypeStruct((1, 1), jnp.float32),
        ],
        scratch_shapes=[
            pltpu.VMEM((N_E, E_DIM), jnp.float32),
            pltpu.VMEM((N_E, 1), jnp.float32),
        ],
    )(z3, embedding_weight, e2t)

    n_el = b * c * h * w
    m = loss_sum[0, 0] / n_el
    loss = m + BETA * m
    return (zq.reshape(b, c, h, w), loss)


# speedup vs baseline: 1.3237x; 1.0115x over previous
import functools

import jax
import jax.numpy as jnp
from jax.experimental import pallas as pl
from jax.experimental.pallas import tpu as pltpu

N_E = 2048
E_DIM = 32
P = 8
BETA = 0.25
HI = 16
LO = 128
GPB = 2


def _vq_block_kernel(z_ref, e_ref, zq_ref, loss_ref, e2_ref, esq_ref,
                     e2t_ref):
    hw = z_ref.shape[2]
    cols = GPB * hw
    n_el = pl.num_programs(0) * E_DIM * cols
    zbf = z_ref[0]
    zb = jnp.concatenate(
        [zbf[g * E_DIM:(g + 1) * E_DIM, :] for g in range(GPB)], axis=1)
    step = pl.program_id(0)

    @pl.when(step == 0)
    def _init():
        e = e_ref[...]
        e2_ref[...] = e + e
        esq_ref[...] = jnp.sum(e * e, axis=1, keepdims=True)
        loss_ref[...] = jnp.zeros((1, 1), jnp.float32)
        for hb in range(HI):
            e2t_ref[hb * E_DIM:(hb + 1) * E_DIM, :] = (
                e[hb * LO:(hb + 1) * LO, :].T.astype(jnp.bfloat16))

    e_sq = esq_ref[...]
    z_sq = jnp.sum(zb * zb, axis=0, keepdims=True)
    mm2 = jax.lax.dot_general(
        e2_ref[...], zb, (((1,), (0,)), ((), ())))
    d = (z_sq + e_sq) - mm2

    dmin = jnp.min(d, axis=0, keepdims=True)
    riota = jax.lax.broadcasted_iota(jnp.int32, (N_E, cols), 0)
    cand = jnp.where(d == dmin, riota, N_E)
    idx = jnp.min(cand, axis=0, keepdims=True)

    lo = idx & (LO - 1)
    hi = idx >> 7
    liota = jax.lax.broadcasted_iota(jnp.int32, (LO, cols), 0)
    ohlo = jnp.where(liota == lo, 1.0, 0.0).astype(jnp.bfloat16)
    t = jax.lax.dot_general(
        e2t_ref[...], ohlo, (((1,), (0,)), ((), ())),
        preferred_element_type=jnp.float32)
    siota = jax.lax.broadcasted_iota(jnp.int32, (HI * E_DIM, cols), 0) >> 5
    pm = jnp.where(siota == hi, t, 0.0)
    acc = pm[0:E_DIM, :]
    for h in range(1, HI):
        acc = acc + pm[h * E_DIM:(h + 1) * E_DIM, :]

    diff = acc - zb
    loss_ref[...] += jnp.sum(diff * diff).reshape(1, 1)

    @pl.when(step == pl.num_programs(0) - 1)
    def _finish():
        loss_ref[...] = loss_ref[...] * ((1.0 + BETA) / n_el)

    out = zb + diff
    zq_ref[...] = jnp.concatenate(
        [out[:, g * hw:(g + 1) * hw] for g in range(GPB)],
        axis=0).reshape(z_ref.shape)


@functools.partial(jax.jit, static_argnames=())
def kernel(z, embedding_weight):
    b, c, h, w = z.shape
    new_c = c // P
    hw = h * w
    grid = (b * P) // GPB
    blocks_per_b = P // GPB
    z3 = z.reshape(b, c, hw)

    zq, loss = pl.pallas_call(
        _vq_block_kernel,
        grid=(grid,),
        in_specs=[
            pl.BlockSpec((1, GPB * new_c, hw),
                         lambda i: (i // blocks_per_b, i % blocks_per_b, 0)),
            pl.BlockSpec((N_E, E_DIM), lambda i: (0, 0)),
        ],
        out_specs=[
            pl.BlockSpec((1, GPB * new_c, hw),
                         lambda i: (i // blocks_per_b, i % blocks_per_b, 0)),
            pl.BlockSpec((1, 1), lambda i: (0, 0)),
        ],
        out_shape=[
            jax.ShapeDtypeStruct((b, c, hw), jnp.float32),
            jax.ShapeDtypeStruct((1, 1), jnp.float32),
        ],
        scratch_shapes=[
            pltpu.VMEM((N_E, E_DIM), jnp.float32),
            pltpu.VMEM((N_E, 1), jnp.float32),
            pltpu.VMEM((HI * E_DIM, LO), jnp.bfloat16),
        ],
    )(z3, embedding_weight)

    return (zq.reshape(b, c, h, w), loss[0, 0])
